# Initial kernel scaffold; baseline (speedup 1.0000x reference)
import jax
import jax.numpy as jnp
from jax.experimental import pallas as pl


def kernel(x, edge_index, W1, b1, W2, b2):
    raise NotImplementedError("write your pallas kernel here")


# speedup vs baseline: 13.7623x; 13.7623x over previous
import functools

import jax
import jax.numpy as jnp
from jax import lax
from jax.experimental import pallas as pl
from jax.experimental.pallas import tpu as pltpu
from jax.experimental.pallas import tpu_sc as plsc

N = 10000
NPAD = 10240
E = 320000
C = 128
NCORES = 2
NSUB = 16
NTILES = NCORES * NSUB
CHUNKS = (E + NTILES * C - 1) // (NTILES * C)
EPAD = NTILES * CHUNKS * C
ROWS_PER_SUB = NPAD // NSUB
IN_CH, HID, OUT_CH = 128, 64, 128


def _make_sc_agg(d):
  mesh = plsc.VectorSubcoreMesh(core_axis_name="c", subcore_axis_name="s")

  @functools.partial(
      pl.kernel,
      out_type=jax.ShapeDtypeStruct((NCORES, NPAD, d), jnp.float32),
      mesh=mesh,
      compiler_params=pltpu.CompilerParams(use_tc_tiling_on_sc=False),
      scratch_types=[
          pltpu.VMEM((CHUNKS, C), jnp.int32),
          pltpu.VMEM((CHUNKS, C), jnp.int32),
          pltpu.VMEM((C, d), jnp.float32),
          pltpu.VMEM_SHARED((NPAD, d), jnp.float32),
      ],
  )
  def agg(table_hbm, src_hbm, dst_hbm, zeros_hbm, out_hbm,
          src_v, dst_v, rows_v, acc_sh):
    c = lax.axis_index("c")
    s = lax.axis_index("s")
    w = c * NSUB + s

    pltpu.sync_copy(zeros_hbm, rows_v)
    row0 = s * ROWS_PER_SUB
    for k in range(ROWS_PER_SUB // C):
      pltpu.sync_copy(rows_v, acc_sh.at[pl.ds(row0 + k * C, C)])

    pltpu.sync_copy(src_hbm.at[w], src_v)
    pltpu.sync_copy(dst_hbm.at[w], dst_v)
    plsc.subcore_barrier()

    @pl.loop(0, CHUNKS)
    def _(j):
      pltpu.sync_copy(table_hbm.at[src_v.at[j]], rows_v)
      pltpu.sync_copy(rows_v, acc_sh.at[dst_v.at[j]], add=True)

    plsc.subcore_barrier()

    for k in range(ROWS_PER_SUB // C):
      sl = pl.ds(row0 + k * C, C)
      pltpu.sync_copy(acc_sh.at[sl], rows_v)
      pltpu.sync_copy(rows_v, out_hbm.at[c, sl])

  return agg


_sc_agg = {d: _make_sc_agg(d) for d in (16, HID, OUT_CH)}


_BM = 1024
_GRID = NPAD // _BM


def _mm_body(x_ref, w_ref, o_ref):
  o_ref[...] = jnp.dot(x_ref[...], w_ref[...],
                       preferred_element_type=jnp.float32)


def _tc_matmul(x, w):
  m, k = x.shape
  n = w.shape[1]
  return pl.pallas_call(
      _mm_body,
      grid=(m // _BM,),
      in_specs=[pl.BlockSpec((_BM, k), lambda i: (i, 0)),
                pl.BlockSpec((k, n), lambda i: (0, 0))],
      out_specs=pl.BlockSpec((_BM, n), lambda i: (i, 0)),
      out_shape=jax.ShapeDtypeStruct((m, n), jnp.float32),
  )(x, w)


def _dinv_scale_body(degp_ref, h_ref, dinv_ref, g_ref, i_ref=None):
  del i_ref
  i = pl.program_id(0)
  deg = degp_ref[0, :, 0:1] + degp_ref[1, :, 0:1] + 1.0
  rid = lax.broadcasted_iota(jnp.int32, (_BM, 1), 0) + i * _BM
  dinv = jnp.where(rid < N, lax.rsqrt(deg), 0.0)
  dinv_ref[...] = dinv
  g_ref[...] = h_ref[...] * dinv


def _tc_dinv_scale(degp, h):
  return pl.pallas_call(
      _dinv_scale_body,
      grid=(_GRID,),
      in_specs=[pl.BlockSpec((NCORES, _BM, 16), lambda i: (0, i, 0)),
                pl.BlockSpec((_BM, HID), lambda i: (i, 0))],
      out_specs=[pl.BlockSpec((_BM, 1), lambda i: (i, 0)),
                 pl.BlockSpec((_BM, HID), lambda i: (i, 0))],
      out_shape=[jax.ShapeDtypeStruct((NPAD, 1), jnp.float32),
                 jax.ShapeDtypeStruct((NPAD, HID), jnp.float32)],
  )(degp, h)


def _mid_body(p_ref, g_ref, dinv_ref, b_ref, w_ref, g2_ref):
  acc = p_ref[0] + p_ref[1] + g_ref[...]
  z = jax.nn.relu(dinv_ref[...] * acc + b_ref[...])
  g2_ref[...] = dinv_ref[...] * jnp.dot(z, w_ref[...],
                                        preferred_element_type=jnp.float32)


def _tc_mid(p, g, dinv, b, w):
  return pl.pallas_call(
      _mid_body,
      grid=(_GRID,),
      in_specs=[pl.BlockSpec((NCORES, _BM, HID), lambda i: (0, i, 0)),
                pl.BlockSpec((_BM, HID), lambda i: (i, 0)),
                pl.BlockSpec((_BM, 1), lambda i: (i, 0)),
                pl.BlockSpec((1, HID), lambda i: (0, 0)),
                pl.BlockSpec((HID, OUT_CH), lambda i: (0, 0))],
      out_specs=pl.BlockSpec((_BM, OUT_CH), lambda i: (i, 0)),
      out_shape=jax.ShapeDtypeStruct((NPAD, OUT_CH), jnp.float32),
  )(p, g, dinv, b, w)


def _final_body(p_ref, g_ref, dinv_ref, b_ref, o_ref):
  acc = p_ref[0] + p_ref[1] + g_ref[...]
  o_ref[...] = jax.nn.relu(dinv_ref[...] * acc + b_ref[...])


def _tc_final(p, g, dinv, b):
  return pl.pallas_call(
      _final_body,
      grid=(_GRID,),
      in_specs=[pl.BlockSpec((NCORES, _BM, OUT_CH), lambda i: (0, i, 0)),
                pl.BlockSpec((_BM, OUT_CH), lambda i: (i, 0)),
                pl.BlockSpec((_BM, 1), lambda i: (i, 0)),
                pl.BlockSpec((1, OUT_CH), lambda i: (0, 0))],
      out_specs=pl.BlockSpec((_BM, OUT_CH), lambda i: (i, 0)),
      out_shape=jax.ShapeDtypeStruct((NPAD, OUT_CH), jnp.float32),
  )(p, g, dinv, b)


def kernel(x, edge_index, W1, b1, W2, b2):
  src = edge_index[0].astype(jnp.int32)
  dst = edge_index[1].astype(jnp.int32)
  pad = jnp.full((EPAD - E,), N, jnp.int32)
  src_p = jnp.concatenate([src, pad]).reshape(NTILES, CHUNKS, C)
  dst_p = jnp.concatenate([dst, pad]).reshape(NTILES, CHUNKS, C)
  x_pad = jnp.zeros((NPAD, IN_CH), jnp.float32).at[:N].set(x)
  ones_table = jnp.zeros((NPAD, 16), jnp.float32).at[:N].set(1.0)
  z16 = jnp.zeros((C, 16), jnp.float32)
  z64 = jnp.zeros((C, HID), jnp.float32)
  z128 = jnp.zeros((C, OUT_CH), jnp.float32)

  degp = _sc_agg[16](ones_table, dst_p, dst_p, z16)
  h1 = _tc_matmul(x_pad, W1)

  dinv, g1 = _tc_dinv_scale(degp, h1)
  p1 = _sc_agg[HID](g1, src_p, dst_p, z64)
  g2 = _tc_mid(p1, g1, dinv, b1.reshape(1, HID), W2)
  p2 = _sc_agg[OUT_CH](g2, src_p, dst_p, z128)
  out = _tc_final(p2, g2, dinv, b2.reshape(1, OUT_CH))
  return out[:N]
